# baseline (speedup 1.0000x reference)
import functools

import jax
import jax.numpy as jnp
from jax import lax
from jax.experimental import pallas as pl
from jax.experimental.pallas import tpu as pltpu
from jax.experimental.pallas import tpu_sc as plsc

_NC = 2
_NS = 16
_L = 16
_NW = _NC * _NS

_SHIFT = 14
_PMASK = (1 << _SHIFT) - 1

_PARAMS = pltpu.CompilerParams(
    needs_layout_passes=False, use_tc_tiling_on_sc=True
)


def _make_k1(M, F, n_u, n_i, mesh):
    NBLK = (M + 127) // 128
    ROWS = n_u + n_i

    @functools.partial(
        pl.kernel,
        mesh=mesh,
        out_type=jax.ShapeDtypeStruct((ROWS + _NW, 128), jnp.float32),
        compiler_params=_PARAMS,
        scratch_types=[
            pltpu.VMEM((n_u,), jnp.int32),
            pltpu.VMEM((n_i,), jnp.int32),
            pltpu.VMEM((n_u + 128,), jnp.int32),
            pltpu.VMEM((n_i + 128,), jnp.int32),
            pltpu.VMEM((n_i + 128,), jnp.int32),
            pltpu.VMEM((6, F, 128), jnp.float32),
            pltpu.VMEM((16, _L, 128), jnp.float32),
            pltpu.SemaphoreType.DMA((6,)),
            pltpu.SemaphoreType.DMA,
        ],
    )
    def k1(u_h, p_h, n_h, uwT_h, iwT_h, stage_h,
           gu, gi, lu, li, lg, blk, rowp, semB, semR):
        wid = lax.axis_index("s") * _NC + lax.axis_index("c")
        start = (NBLK * wid) // _NW
        nb = (NBLK * (wid + 1)) // _NW - start

        pltpu.sync_copy(u_h, gu)
        pltpu.sync_copy(p_h, gi.at[pl.ds(0, n_i // 2)])
        pltpu.sync_copy(n_h, gi.at[pl.ds(n_i // 2, n_i // 2)])

        lanes = lax.iota(jnp.int32, _L)

        def compact(src, dst, nchunks, pos0, nin):
            assert nchunks % 2 == 0

            def body(j, n):
                for h in range(2):
                    kk = 2 * j + h
                    raw = src[pl.ds(kk * _L, _L)]
                    v = (raw << _SHIFT) + (pos0 + kk * _L + lanes)
                    b = raw >> 7
                    m = (b >= start) & (b < start + nb)
                    plsc.store_compressed(dst.at[pl.ds(n, _L)], v, mask=m)
                    pc = plsc.all_reduce_population_count(m)
                    n = n + pc[0]
                return n
            return lax.fori_loop(nin, nchunks // 2, body, jnp.int32(0))

        ncu = compact(gu, lu, n_u // _L, 0, 0)
        nci = compact(gi, li, n_i // _L, n_u, 0)

        def fetch(tbl, c, slot):
            cc = jnp.minimum(c, NBLK - 1)
            off = pl.multiple_of(cc * 128, 128)
            pltpu.async_copy(
                tbl.at[:, pl.ds(off, 128)], blk.at[slot], semB.at[slot]
            )

        def wait_block(slot):
            pltpu.make_async_copy(
                uwT_h.at[:, pl.ds(0, 128)], blk.at[slot], semB.at[slot]
            ).wait()

        def process_table(tbl, lst, cnt, kcin):
            nchunk = (cnt + _L - 1) // _L
            for r in range(5):
                fetch(tbl, start + r, r)

            def block_body(j, kc0):
                slot = j % 6
                wait_block(slot)

                @pl.when(j + 5 < nb)
                def _():
                    fetch(tbl, start + j + 5, (j + 5) % 6)

                c = start + j

                def cbody(k, n):
                    for h in range(2):
                        kk = 2 * k + h
                        v = lst[pl.ds(kk * _L, _L)]
                        valid = (kk * _L + lanes) < cnt
                        m = ((v >> (_SHIFT + 7)) == c) & valid
                        plsc.store_compressed(lg.at[pl.ds(n, _L)], v,
                                              mask=m)
                        pc = plsc.all_reduce_population_count(m)
                        n = n + pc[0]
                    return n

                nh = lax.fori_loop(0, (nchunk + 1) // 2, cbody,
                                   jnp.int32(0))

                def hbody(k, kc):
                    v = lg[pl.ds(k * _L, _L)]
                    valid = (k * _L + lanes) < nh
                    cols = (v >> _SHIFT) & 127
                    pv = v & _PMASK
                    grp = kc % 16
                    grpv = jnp.full((_L,), grp, jnp.int32)
                    slotv = jnp.full((_L,), slot, jnp.int32)
                    for f in range(F):
                        fv = jnp.full((_L,), f, jnp.int32)
                        vals = plsc.load_gather(blk, [slotv, fv, cols])
                        didx = (f + pv) & (F - 1)
                        plsc.store_scatter(rowp, [grpv, lanes, didx], vals)
                    pfix = jnp.where(valid, pv, ROWS + wid)
                    pltpu.async_copy(rowp.at[grp], stage_h.at[pfix], semR)

                    @pl.when(kc % 16 == 15)
                    def _():
                        for _i in range(16):
                            pltpu.make_async_copy(
                                rowp.at[0], stage_h.at[pfix], semR
                            ).wait()

                    return kc + 1

                return lax.fori_loop(0, (nh + _L - 1) // _L, hbody, kc0)

            return lax.fori_loop(0, nb, block_body, kcin)

        kc = process_table(uwT_h, lu, ncu, jnp.int32(0))
        kc = process_table(iwT_h, li, nci, kc)

        trash = jnp.full((_L,), ROWS + wid, jnp.int32)

        def drain_body(_, x):
            pltpu.make_async_copy(
                rowp.at[0], stage_h.at[trash], semR
            ).wait()
            return x
        lax.fori_loop(0, kc % 16, drain_body, jnp.int32(0))

    return k1


def _make_k2(B, F, mesh):
    bpw = B // _NW
    CH = bpw * 128

    @functools.partial(
        pl.kernel,
        mesh=mesh,
        out_type=jax.ShapeDtypeStruct((B,), jnp.float32),
        compiler_params=_PARAMS,
        scratch_types=[
            pltpu.VMEM((bpw, 128), jnp.float32),
            pltpu.VMEM((bpw, 128), jnp.float32),
            pltpu.VMEM((bpw, 128), jnp.float32),
            pltpu.VMEM((bpw,), jnp.float32),
            pltpu.SemaphoreType.DMA,
        ],
    )
    def k2(stage_h, out_h, ur, pr, nr, ov, sem):
        wid = lax.axis_index("s") * _NC + lax.axis_index("c")
        base = wid * bpw

        cu = pltpu.async_copy(stage_h.at[pl.ds(base, bpw)], ur, sem)
        cp = pltpu.async_copy(stage_h.at[pl.ds(B + base, bpw)], pr, sem)
        cn = pltpu.async_copy(stage_h.at[pl.ds(2 * B + base, bpw)], nr, sem)
        cu.wait()
        cp.wait()
        cn.wait()

        lanes = lax.iota(jnp.int32, _L)

        def group(g, carry):
            rows = g * _L + lanes
            rot = g * _L + lanes
            acc = jnp.zeros((_L,), jnp.float32)
            for f in range(F):
                col = (f + rot) & (F - 1)
                uu = plsc.load_gather(ur, [rows, col])
                pp = plsc.load_gather(pr, [rows, col])
                nn = plsc.load_gather(nr, [rows, col])
                acc = acc + uu * (pp - nn)
            sig = 1.0 / (1.0 + jnp.exp(-acc))
            ov[pl.ds(g * _L, _L)] = 2.0 - sig
            return carry

        lax.fori_loop(0, bpw // _L, group, 0)
        pltpu.sync_copy(ov, out_h.at[pl.ds(pl.multiple_of(wid * bpw, 128),
                                           bpw)])

    return k2


def kernel(user, posItem, negItem, user_W, item_W):
    B = user.shape[0]
    M, F = user_W.shape
    n_i = 2 * B
    mesh = plsc.VectorSubcoreMesh(core_axis_name="c", subcore_axis_name="s")
    stage = _make_k1(M, F, B, n_i, mesh)(user, posItem, negItem,
                                         user_W.T, item_W.T)
    out = _make_k2(B, F, mesh)(stage)
    return out.reshape(-1, 1)
